# Initial kernel scaffold; baseline (speedup 1.0000x reference)
import jax
import jax.numpy as jnp
from jax.experimental import pallas as pl


def kernel(P, P0, z_3d, z_text):
    raise NotImplementedError("write your pallas kernel here")


# speedup vs baseline: 25.3573x; 25.3573x over previous
import functools
import jax
import jax.numpy as jnp
from jax import lax
from jax.experimental import pallas as pl
from jax.experimental.pallas import tpu as pltpu

_K = 8
_LAMBDA_TEXT = 0.5
_LAMBDA_STICK = 2.0
_LAMBDA_SMOOTH = 0.1


def _loss_body(p_rows_ref, pt_ref, p0t_ref, z3_ref, zt_ref,
               fwd_ref, bwd_ref, smooth_ref, text_ref, colmin_ref):
    t = pl.program_id(1)
    T = pl.num_programs(1)
    p_tile = p_rows_ref[0]
    pt = pt_ref[0]
    p0t = p0t_ref[0]
    R = p_tile.shape[0]
    N = pt.shape[1]

    px = p_tile[:, 0:1]
    py = p_tile[:, 1:2]
    pz = p_tile[:, 2:3]

    dx = px - p0t[0:1, :]
    dy = py - p0t[1:2, :]
    dz = pz - p0t[2:3, :]
    d0 = dx * dx + dy * dy + dz * dz
    fwd_ref[0, 0, 0, 0] = jnp.sum(jnp.min(d0, axis=1))
    cm = jnp.min(d0, axis=0, keepdims=True)

    @pl.when(t == 0)
    def _():
        colmin_ref[...] = cm

    @pl.when(t > 0)
    def _():
        colmin_ref[...] = jnp.minimum(colmin_ref[...], cm)

    bwd_ref[0, 0, 0, 0] = jnp.where(t == T - 1, jnp.sum(colmin_ref[...]), 0.0)

    sx = px - pt[0:1, :]
    sy = py - pt[1:2, :]
    sz = pz - pt[2:3, :]
    ds = sx * sx + sy * sy + sz * sz
    row_ids = t * R + lax.broadcasted_iota(jnp.int32, (R, N), 0)
    col_ids = lax.broadcasted_iota(jnp.int32, (R, N), 1)
    ds = jnp.where(row_ids == col_ids, ds + 1.0e6, ds)

    work = ds
    for _ in range(_K - 1):
        m = jnp.min(work, axis=1, keepdims=True)
        work = jnp.where(work <= m, jnp.inf, work)
    t8 = jnp.min(work, axis=1, keepdims=True)

    maskf = (ds <= t8).astype(jnp.float32)
    count = jnp.sum(maskf, axis=1, keepdims=True)
    nx = jnp.sum(maskf * pt[0:1, :], axis=1, keepdims=True)
    ny = jnp.sum(maskf * pt[1:2, :], axis=1, keepdims=True)
    nz = jnp.sum(maskf * pt[2:3, :], axis=1, keepdims=True)
    ex = px - nx / count
    ey = py - ny / count
    ez = pz - nz / count
    smooth_ref[0, 0, 0, 0] = jnp.sum(ex * ex + ey * ey + ez * ez)

    z3 = z3_ref[...]
    zt = zt_ref[...]
    n3 = jnp.maximum(jnp.sqrt(jnp.sum(z3 * z3)), 1.0e-12)
    nt = jnp.maximum(jnp.sqrt(jnp.sum(zt * zt)), 1.0e-12)
    text_ref[0, 0, 0, 0] = jnp.sum(z3 * zt) / (n3 * nt)


@jax.jit
def kernel(P, P0, z_3d, z_text):
    B, N, _ = P.shape
    R = 512
    T = N // R
    PT = P.transpose(0, 2, 1)
    P0T = P0.transpose(0, 2, 1)

    grid = (B, T)
    out_shapes = [jax.ShapeDtypeStruct((B, T, 1, 1), jnp.float32)] * 4
    scalar_spec = pl.BlockSpec((1, 1, 1, 1), lambda b, t: (b, t, 0, 0),
                               memory_space=pltpu.SMEM)
    fwd, bwd, smooth, text = pl.pallas_call(
        _loss_body,
        grid=grid,
        in_specs=[
            pl.BlockSpec((1, R, 3), lambda b, t: (b, t, 0)),
            pl.BlockSpec((1, 3, N), lambda b, t: (b, 0, 0)),
            pl.BlockSpec((1, 3, N), lambda b, t: (b, 0, 0)),
            pl.BlockSpec((1, 1, z_3d.shape[1]), lambda b, t: (b, 0, 0)),
            pl.BlockSpec((1, 1, z_text.shape[1]), lambda b, t: (b, 0, 0)),
        ],
        out_specs=[scalar_spec] * 4,
        out_shape=out_shapes,
        scratch_shapes=[pltpu.VMEM((1, N), jnp.float32)],
        compiler_params=pltpu.CompilerParams(
            dimension_semantics=("arbitrary", "arbitrary"),
        ),
    )(P, PT, P0T, z_3d[:, None, :], z_text[:, None, :])

    inv = 1.0 / (B * N)
    L_stick = (jnp.sum(fwd) + jnp.sum(bwd)) * inv
    L_smooth = jnp.sum(smooth) * (inv / 3.0)
    L_text = -jnp.mean(text[:, 0, 0, 0])
    L_total = (_LAMBDA_TEXT * L_text + _LAMBDA_STICK * L_stick
               + _LAMBDA_SMOOTH * L_smooth)
    return jnp.stack([L_total, L_text, L_stick, L_smooth])
